# baseline (speedup 1.0000x reference)
import functools

import jax
import jax.numpy as jnp
from jax import lax
from jax.experimental import pallas as pl
from jax.experimental.pallas import tpu as pltpu
from jax.experimental.pallas import tpu_sc as plsc

_N = 10000
_E = 320000
_D = 128
_NC = 2
_NS = 16
_NW = _NC * _NS
_NPAD = 10240
_SEG = _NPAD // _NS
_EPS = _E // _NS
_EPT = _E // _NW
_C = 128
_NCH = 80
_EP = _NW * _NCH * _C
_NG = _NCH // 2
_CF = _D // _NC
_NROW = _N // _NS

_mesh = plsc.VectorSubcoreMesh(core_axis_name="c", subcore_axis_name="s")
_sc_params = pltpu.CompilerParams(needs_layout_passes=False,
                                  use_tc_tiling_on_sc=False)


def _rsqrt16(x):
    i = plsc.bitcast(x, jnp.int32)
    i = jnp.int32(0x5F3759DF) - jnp.right_shift(i, 1)
    y = plsc.bitcast(i, jnp.float32)
    for _ in range(4):
        y = y * (jnp.float32(1.5) - jnp.float32(0.5) * x * y * y)
    return y


@functools.partial(
    pl.kernel,
    mesh=_mesh,
    out_type=(
        jax.ShapeDtypeStruct((_NPAD,), jnp.float32),
        jax.ShapeDtypeStruct((_E,), jnp.float32),
    ),
    scratch_types=[
        pltpu.VMEM((_EPS,), jnp.int32),
        pltpu.VMEM((_EPS,), jnp.float32),
        pltpu.VMEM((_NPAD,), jnp.float32),
        pltpu.VMEM((_SEG,), jnp.float32),
        pltpu.VMEM((_SEG,), jnp.float32),
        pltpu.VMEM((_NPAD,), jnp.float32),
        pltpu.VMEM((_EPT,), jnp.int32),
        pltpu.VMEM((_EPT,), jnp.float32),
        pltpu.VMEM_SHARED((_NS, _NPAD), jnp.float32),
        pltpu.VMEM_SHARED((_NPAD,), jnp.float32),
    ],
    compiler_params=_sc_params,
)
def _norm_kernel(src_hbm, dst_hbm, w_hbm, dinv2_hbm, norm_hbm,
                 dst_v, w_v, deg_v, acc_v, tmp_v, dinv_v, src_v, nrm_v,
                 slab_sh, dinv_sh):
    cid = lax.axis_index("c")
    sid = lax.axis_index("s")
    wid = cid * _NS + sid

    ebase = sid * _EPS
    pltpu.sync_copy(dst_hbm.at[pl.ds(ebase, _EPS)], dst_v)
    pltpu.sync_copy(w_hbm.at[pl.ds(ebase, _EPS)], w_v)

    def _zero(i, _):
        deg_v[pl.ds(i * 16, 16)] = jnp.zeros((16,), jnp.float32)
        return _
    lax.fori_loop(0, _NPAD // 16, _zero, None)

    def _deg(i, _):
        d16 = dst_v[pl.ds(i * 16, 16)]
        w16 = w_v[pl.ds(i * 16, 16)]
        plsc.addupdate_scatter(deg_v, [d16], w16)
        return _
    lax.fori_loop(0, _EPS // 16, _deg, None)

    pltpu.sync_copy(deg_v, slab_sh.at[sid])
    plsc.subcore_barrier()

    col0 = sid * _SEG
    pltpu.sync_copy(slab_sh.at[0, pl.ds(col0, _SEG)], acc_v)

    def _red(r, _):
        pltpu.sync_copy(slab_sh.at[r, pl.ds(col0, _SEG)], tmp_v)

        def _add(k, __):
            acc_v[pl.ds(k * 16, 16)] = (acc_v[pl.ds(k * 16, 16)]
                                        + tmp_v[pl.ds(k * 16, 16)])
            return __
        lax.fori_loop(0, _SEG // 16, _add, None)
        return _
    lax.fori_loop(1, _NS, _red, None)

    def _dinv(k, _):
        d = acc_v[pl.ds(k * 16, 16)] + jnp.float32(1.0)
        y = _rsqrt16(d)
        acc_v[pl.ds(k * 16, 16)] = y
        tmp_v[pl.ds(k * 16, 16)] = y * y
        return _
    lax.fori_loop(0, _SEG // 16, _dinv, None)

    pltpu.sync_copy(acc_v, dinv_sh.at[pl.ds(col0, _SEG)])

    @pl.when(cid == 0)
    def _():
        pltpu.sync_copy(tmp_v, dinv2_hbm.at[pl.ds(col0, _SEG)])

    plsc.subcore_barrier()
    pltpu.sync_copy(dinv_sh, dinv_v)

    nbase = wid * _EPT
    pltpu.sync_copy(src_hbm.at[pl.ds(nbase, _EPT)], src_v)
    pltpu.sync_copy(dst_hbm.at[pl.ds(nbase, _EPT)], dst_v.at[pl.ds(0, _EPT)])
    pltpu.sync_copy(w_hbm.at[pl.ds(nbase, _EPT)], w_v.at[pl.ds(0, _EPT)])

    def _nrm(i, _):
        s16 = src_v[pl.ds(i * 16, 16)]
        d16 = dst_v[pl.ds(i * 16, 16)]
        w16 = w_v[pl.ds(i * 16, 16)]
        a = plsc.load_gather(dinv_v, [s16])
        b = plsc.load_gather(dinv_v, [d16])
        nrm_v[pl.ds(i * 16, 16)] = a * w16 * b
        return _
    lax.fori_loop(0, _EPT // 16, _nrm, None)

    pltpu.sync_copy(nrm_v, norm_hbm.at[pl.ds(nbase, _EPT)])


@functools.partial(
    pl.kernel,
    mesh=_mesh,
    out_type=jax.ShapeDtypeStruct((_NC, _N, _D), jnp.float32),
    scratch_types=[
        pltpu.VMEM((_NCH, _C), jnp.int32),
        pltpu.VMEM((2, _C, _D), jnp.float32),
        pltpu.VMEM((2, _C), jnp.int32),
        pltpu.VMEM((2, _C), jnp.int32),
        pltpu.VMEM((2, _C), jnp.float32),
        pltpu.VMEM_SHARED((_N, _D), jnp.float32),
        pltpu.SemaphoreType.DMA,
        pltpu.SemaphoreType.DMA,
        pltpu.SemaphoreType.DMA,
        pltpu.SemaphoreType.DMA,
    ],
    compiler_params=_sc_params,
)
def _agg_kernel(h_hbm, pk_hbm, norm_hbm, out_hbm,
                pk_v, rows_v, src_r, dst_r, nrm_r, acc_sh,
                g0, g1, n0, n1):
    gsem = (g0, g1)
    nsem = (n0, n1)
    cid = lax.axis_index("c")
    sid = lax.axis_index("s")
    wid = cid * _NS + sid
    row0 = wid * _NCH

    def _unpack(cc, slot):
        for k in range(_C // 16):
            p16 = pk_v[cc, pl.ds(k * 16, 16)]
            src_r[slot, pl.ds(k * 16, 16)] = p16 & jnp.int32(0x3FFF)
            dst_r[slot, pl.ds(k * 16, 16)] = jnp.right_shift(p16, 14)

    def _nissue(cc, slot):
        pltpu.async_copy(norm_hbm.at[row0 + cc], nrm_r.at[slot], nsem[slot])

    def _nwait(cc, slot):
        pltpu.make_async_copy(norm_hbm.at[row0 + cc], nrm_r.at[slot],
                              nsem[slot]).wait()

    def _gissue(b):
        pltpu.async_copy(h_hbm.at[src_r.at[b]], rows_v.at[b], gsem[b])

    def _gwait(b):
        pltpu.make_async_copy(h_hbm.at[src_r.at[b]], rows_v.at[b],
                              gsem[b]).wait()

    def _zr(i, _):
        for k in range(_D // 16):
            rows_v[0, i, pl.ds(k * 16, 16)] = jnp.zeros((16,), jnp.float32)
        return _
    lax.fori_loop(0, _C, _zr, None)

    r0 = sid * _NROW
    for j in range(_NROW // _C):
        pltpu.sync_copy(rows_v.at[0], acc_sh.at[pl.ds(r0 + j * _C, _C)])
    rem = _NROW % _C
    if rem:
        pltpu.sync_copy(rows_v.at[0, pl.ds(0, rem)],
                        acc_sh.at[pl.ds(r0 + (_NROW // _C) * _C, rem)])

    pltpu.sync_copy(pk_hbm.at[pl.ds(row0, _NCH)], pk_v)
    plsc.subcore_barrier()

    _unpack(0, 0)
    _nissue(0, 0)
    _gissue(0)

    def _group(g, _):
        for u in range(2):
            c = g * 2 + u
            b = u
            nb_ = 1 - u
            if u == 0:
                _unpack(c + 1, nb_)
                _nissue(c + 1, nb_)
                _gissue(nb_)
            else:
                @pl.when(g < _NG - 1)
                def _():
                    _unpack(c + 1, nb_)
                    _nissue(c + 1, nb_)
                    _gissue(nb_)
            _gwait(b)
            _nwait(c, b)

            def _scale(grp, __, b=b):
                n16 = nrm_r[b, pl.ds(grp * 16, 16)]
                for l in range(16):
                    e = grp * 16 + l
                    nbv = jnp.broadcast_to(n16[l], (16,))
                    for k in range(_D // 16):
                        rows_v[b, e, pl.ds(k * 16, 16)] = (
                            rows_v[b, e, pl.ds(k * 16, 16)] * nbv)
                return __
            lax.fori_loop(0, _C // 16, _scale, None)

            pltpu.sync_copy(rows_v.at[b], acc_sh.at[dst_r.at[b]], add=True)
        return _
    lax.fori_loop(0, _NG, _group, None)

    plsc.subcore_barrier()
    pltpu.sync_copy(acc_sh.at[pl.ds(r0, _NROW)],
                    out_hbm.at[cid, pl.ds(r0, _NROW)])


_BLK = 400


def _mm_body(x_ref, w_ref, o_ref):
    o_ref[...] = jnp.dot(x_ref[...], w_ref[...],
                         preferred_element_type=jnp.float32)


def _matmul(x, w):
    return pl.pallas_call(
        _mm_body,
        grid=(_N // _BLK,),
        in_specs=[
            pl.BlockSpec((_BLK, _D), lambda i: (i, 0)),
            pl.BlockSpec((_D, _D), lambda i: (0, 0)),
        ],
        out_specs=pl.BlockSpec((_BLK, _D), lambda i: (i, 0)),
        out_shape=jax.ShapeDtypeStruct((_N, _D), jnp.float32),
    )(x, w)


def _mid_body(p_ref, h_ref, d_ref, b_ref, w_ref, o_ref):
    agg = p_ref[0] + p_ref[1] + d_ref[...] * h_ref[...] + b_ref[...]
    a = jnp.maximum(agg, 0.0)
    o_ref[...] = jnp.dot(a, w_ref[...], preferred_element_type=jnp.float32)


def _mid(p, h, dinv2, b, w):
    return pl.pallas_call(
        _mid_body,
        grid=(_N // _BLK,),
        in_specs=[
            pl.BlockSpec((_NC, _BLK, _D), lambda i: (0, i, 0)),
            pl.BlockSpec((_BLK, _D), lambda i: (i, 0)),
            pl.BlockSpec((_BLK, 1), lambda i: (i, 0)),
            pl.BlockSpec((1, _D), lambda i: (0, 0)),
            pl.BlockSpec((_D, _D), lambda i: (0, 0)),
        ],
        out_specs=pl.BlockSpec((_BLK, _D), lambda i: (i, 0)),
        out_shape=jax.ShapeDtypeStruct((_N, _D), jnp.float32),
    )(p, h, dinv2, b.reshape(1, _D), w)


def _final_body(p_ref, h_ref, d_ref, b_ref, o_ref):
    agg = p_ref[0] + p_ref[1] + d_ref[...] * h_ref[...] + b_ref[...]
    o_ref[...] = jax.nn.sigmoid(agg)


def _final(p, h, dinv2, b):
    return pl.pallas_call(
        _final_body,
        grid=(_N // _BLK,),
        in_specs=[
            pl.BlockSpec((_NC, _BLK, _D), lambda i: (0, i, 0)),
            pl.BlockSpec((_BLK, _D), lambda i: (i, 0)),
            pl.BlockSpec((_BLK, 1), lambda i: (i, 0)),
            pl.BlockSpec((1, _D), lambda i: (0, 0)),
        ],
        out_specs=pl.BlockSpec((_BLK, _D), lambda i: (i, 0)),
        out_shape=jax.ShapeDtypeStruct((_N, _D), jnp.float32),
    )(p, h, dinv2, b.reshape(1, _D))


def kernel(x, edge_index, edge_weights, W1, b1, W2, b2, W3, b3):
    src = edge_index[0]
    dst = edge_index[1]

    dinv2_pad, norm = _norm_kernel(src, dst, edge_weights)
    dinv2 = dinv2_pad[:_N].reshape(_N, 1)

    pad = _EP - _E
    packed = jnp.bitwise_or(src, jnp.left_shift(dst, 14))
    pk2d = jnp.concatenate(
        [packed, jnp.zeros((pad,), jnp.int32)]).reshape(_EP // _C, _C)
    norm2d = jnp.concatenate(
        [norm, jnp.zeros((pad,), jnp.float32)]).reshape(_EP // _C, _C)

    h1 = _matmul(x, W1)
    p1 = _agg_kernel(h1, pk2d, norm2d)
    h2 = _mid(p1, h1, dinv2, b1, W2)
    p2 = _agg_kernel(h2, pk2d, norm2d)
    h3 = _mid(p2, h2, dinv2, b2, W3)
    p3 = _agg_kernel(h3, pk2d, norm2d)
    return _final(p3, h3, dinv2, b3)


# speedup vs baseline: 1.8434x; 1.8434x over previous
import functools

import jax
import jax.numpy as jnp
from jax import lax
from jax.experimental import pallas as pl
from jax.experimental.pallas import tpu as pltpu
from jax.experimental.pallas import tpu_sc as plsc

_N = 10000
_E = 320000
_D = 128
_NC = 2
_NS = 16
_NW = _NC * _NS
_NPAD = 10240
_SEG = _NPAD // _NS
_EPS = _E // _NS
_EPT = _E // _NW
_C = 80
_NCH = 126
_EP = _NW * _NCH * _C
_NG = _NCH // 2
_NROW = _N // _NS

_mesh = plsc.VectorSubcoreMesh(core_axis_name="c", subcore_axis_name="s")
_sc_params = pltpu.CompilerParams(needs_layout_passes=False,
                                  use_tc_tiling_on_sc=False)


def _rsqrt16(x):
    i = plsc.bitcast(x, jnp.int32)
    i = jnp.int32(0x5F3759DF) - jnp.right_shift(i, 1)
    y = plsc.bitcast(i, jnp.float32)
    for _ in range(4):
        y = y * (jnp.float32(1.5) - jnp.float32(0.5) * x * y * y)
    return y


@functools.partial(
    pl.kernel,
    mesh=_mesh,
    out_type=(
        jax.ShapeDtypeStruct((_NPAD,), jnp.float32),
        jax.ShapeDtypeStruct((_E,), jnp.float32),
    ),
    scratch_types=[
        pltpu.VMEM((_EPS,), jnp.int32),
        pltpu.VMEM((_EPS,), jnp.float32),
        pltpu.VMEM((_NPAD,), jnp.float32),
        pltpu.VMEM((_SEG,), jnp.float32),
        pltpu.VMEM((_SEG,), jnp.float32),
        pltpu.VMEM((_NPAD,), jnp.float32),
        pltpu.VMEM((_EPT,), jnp.int32),
        pltpu.VMEM((_EPT,), jnp.float32),
        pltpu.VMEM_SHARED((_NS, _NPAD), jnp.float32),
        pltpu.VMEM_SHARED((_NPAD,), jnp.float32),
    ],
    compiler_params=_sc_params,
)
def _norm_kernel(src_hbm, dst_hbm, w_hbm, dinv2_hbm, norm_hbm,
                 dst_v, w_v, deg_v, acc_v, tmp_v, dinv_v, src_v, nrm_v,
                 slab_sh, dinv_sh):
    cid = lax.axis_index("c")
    sid = lax.axis_index("s")
    wid = cid * _NS + sid

    ebase = sid * _EPS
    pltpu.sync_copy(dst_hbm.at[pl.ds(ebase, _EPS)], dst_v)
    pltpu.sync_copy(w_hbm.at[pl.ds(ebase, _EPS)], w_v)

    def _zero(i, _):
        deg_v[pl.ds(i * 16, 16)] = jnp.zeros((16,), jnp.float32)
        return _
    lax.fori_loop(0, _NPAD // 16, _zero, None)

    def _deg(i, _):
        d16 = dst_v[pl.ds(i * 16, 16)]
        w16 = w_v[pl.ds(i * 16, 16)]
        plsc.addupdate_scatter(deg_v, [d16], w16)
        return _
    lax.fori_loop(0, _EPS // 16, _deg, None)

    pltpu.sync_copy(deg_v, slab_sh.at[sid])
    plsc.subcore_barrier()

    col0 = sid * _SEG
    pltpu.sync_copy(slab_sh.at[0, pl.ds(col0, _SEG)], acc_v)

    def _red(r, _):
        pltpu.sync_copy(slab_sh.at[r, pl.ds(col0, _SEG)], tmp_v)

        def _add(k, __):
            acc_v[pl.ds(k * 16, 16)] = (acc_v[pl.ds(k * 16, 16)]
                                        + tmp_v[pl.ds(k * 16, 16)])
            return __
        lax.fori_loop(0, _SEG // 16, _add, None)
        return _
    lax.fori_loop(1, _NS, _red, None)

    def _dinv(k, _):
        d = acc_v[pl.ds(k * 16, 16)] + jnp.float32(1.0)
        y = _rsqrt16(d)
        acc_v[pl.ds(k * 16, 16)] = y
        tmp_v[pl.ds(k * 16, 16)] = y * y
        return _
    lax.fori_loop(0, _SEG // 16, _dinv, None)

    pltpu.sync_copy(acc_v, dinv_sh.at[pl.ds(col0, _SEG)])

    @pl.when(cid == 0)
    def _():
        pltpu.sync_copy(tmp_v, dinv2_hbm.at[pl.ds(col0, _SEG)])

    plsc.subcore_barrier()
    pltpu.sync_copy(dinv_sh, dinv_v)

    nbase = wid * _EPT
    pltpu.sync_copy(src_hbm.at[pl.ds(nbase, _EPT)], src_v)
    pltpu.sync_copy(dst_hbm.at[pl.ds(nbase, _EPT)], dst_v.at[pl.ds(0, _EPT)])
    pltpu.sync_copy(w_hbm.at[pl.ds(nbase, _EPT)], w_v.at[pl.ds(0, _EPT)])

    def _nrm(i, _):
        s16 = src_v[pl.ds(i * 16, 16)]
        d16 = dst_v[pl.ds(i * 16, 16)]
        w16 = w_v[pl.ds(i * 16, 16)]
        a = plsc.load_gather(dinv_v, [s16])
        b = plsc.load_gather(dinv_v, [d16])
        nrm_v[pl.ds(i * 16, 16)] = a * w16 * b
        return _
    lax.fori_loop(0, _EPT // 16, _nrm, None)

    pltpu.sync_copy(nrm_v, norm_hbm.at[pl.ds(nbase, _EPT)])


@functools.partial(
    pl.kernel,
    mesh=_mesh,
    out_type=jax.ShapeDtypeStruct((_NC, _N, _D), jnp.float32),
    scratch_types=[
        pltpu.VMEM((_NCH, _C), jnp.int32),
        pltpu.VMEM((_NCH, _C), jnp.int32),
        pltpu.VMEM((_NCH, _C), jnp.float32),
        pltpu.VMEM((2, _C, _D), jnp.float32),
        pltpu.VMEM_SHARED((_N, _D), jnp.float32),
        pltpu.SemaphoreType.DMA,
        pltpu.SemaphoreType.DMA,
    ],
    compiler_params=_sc_params,
)
def _agg_kernel(h_hbm, src_hbm, dst_hbm, norm_hbm, out_hbm,
                src_v, dst_v, norm_v, rows_v, acc_sh, g0, g1):
    gsem = (g0, g1)
    cid = lax.axis_index("c")
    sid = lax.axis_index("s")
    wid = cid * _NS + sid
    row0 = wid * _NCH

    def _gissue(cc, b):
        pltpu.async_copy(h_hbm.at[src_v.at[cc]], rows_v.at[b], gsem[b])

    def _gwait(cc, b):
        pltpu.make_async_copy(h_hbm.at[src_v.at[cc]], rows_v.at[b],
                              gsem[b]).wait()

    def _zr(i, _):
        for k in range(_D // 16):
            rows_v[0, i, pl.ds(k * 16, 16)] = jnp.zeros((16,), jnp.float32)
        return _
    lax.fori_loop(0, _C, _zr, None)

    r0 = sid * _NROW
    for j in range(_NROW // _C):
        pltpu.sync_copy(rows_v.at[0], acc_sh.at[pl.ds(r0 + j * _C, _C)])
    rem = _NROW % _C
    if rem:
        pltpu.sync_copy(rows_v.at[0, pl.ds(0, rem)],
                        acc_sh.at[pl.ds(r0 + (_NROW // _C) * _C, rem)])

    pltpu.sync_copy(src_hbm.at[pl.ds(row0, _NCH)], src_v)
    pltpu.sync_copy(dst_hbm.at[pl.ds(row0, _NCH)], dst_v)
    pltpu.sync_copy(norm_hbm.at[pl.ds(row0, _NCH)], norm_v)
    plsc.subcore_barrier()

    _gissue(0, 0)

    def _compute(c, b):
        def _scale(grp, __):
            n16 = norm_v[c, pl.ds(grp * 16, 16)]
            for l in range(16):
                e = grp * 16 + l
                nb = jnp.broadcast_to(n16[l], (16,))
                for k in range(_D // 16):
                    rows_v[b, e, pl.ds(k * 16, 16)] = (
                        rows_v[b, e, pl.ds(k * 16, 16)] * nb)
            return __
        lax.fori_loop(0, _C // 16, _scale, None)

    def _group(g, _):
        for u in range(2):
            c = g * 2 + u
            b = u
            if u == 0:
                _gissue(c + 1, 1)
            else:
                @pl.when(g < _NG - 1)
                def _():
                    _gissue(c + 1, 0)
            _gwait(c, b)
            _compute(c, b)
            pltpu.sync_copy(rows_v.at[b], acc_sh.at[dst_v.at[c]], add=True)
        return _
    lax.fori_loop(0, _NG, _group, None)

    plsc.subcore_barrier()
    pltpu.sync_copy(acc_sh.at[pl.ds(r0, _NROW)],
                    out_hbm.at[cid, pl.ds(r0, _NROW)])


_BLK = 400


def _mm_body(x_ref, w_ref, o_ref):
    o_ref[...] = jnp.dot(x_ref[...], w_ref[...],
                         preferred_element_type=jnp.float32)


def _matmul(x, w):
    return pl.pallas_call(
        _mm_body,
        grid=(_N // _BLK,),
        in_specs=[
            pl.BlockSpec((_BLK, _D), lambda i: (i, 0)),
            pl.BlockSpec((_D, _D), lambda i: (0, 0)),
        ],
        out_specs=pl.BlockSpec((_BLK, _D), lambda i: (i, 0)),
        out_shape=jax.ShapeDtypeStruct((_N, _D), jnp.float32),
    )(x, w)


def _mid_body(p_ref, h_ref, d_ref, b_ref, w_ref, o_ref):
    agg = p_ref[0] + p_ref[1] + d_ref[...] * h_ref[...] + b_ref[...]
    a = jnp.maximum(agg, 0.0)
    o_ref[...] = jnp.dot(a, w_ref[...], preferred_element_type=jnp.float32)


def _mid(p, h, dinv2, b, w):
    return pl.pallas_call(
        _mid_body,
        grid=(_N // _BLK,),
        in_specs=[
            pl.BlockSpec((_NC, _BLK, _D), lambda i: (0, i, 0)),
            pl.BlockSpec((_BLK, _D), lambda i: (i, 0)),
            pl.BlockSpec((_BLK, 1), lambda i: (i, 0)),
            pl.BlockSpec((1, _D), lambda i: (0, 0)),
            pl.BlockSpec((_D, _D), lambda i: (0, 0)),
        ],
        out_specs=pl.BlockSpec((_BLK, _D), lambda i: (i, 0)),
        out_shape=jax.ShapeDtypeStruct((_N, _D), jnp.float32),
    )(p, h, dinv2, b.reshape(1, _D), w)


def _final_body(p_ref, h_ref, d_ref, b_ref, o_ref):
    agg = p_ref[0] + p_ref[1] + d_ref[...] * h_ref[...] + b_ref[...]
    o_ref[...] = jax.nn.sigmoid(agg)


def _final(p, h, dinv2, b):
    return pl.pallas_call(
        _final_body,
        grid=(_N // _BLK,),
        in_specs=[
            pl.BlockSpec((_NC, _BLK, _D), lambda i: (0, i, 0)),
            pl.BlockSpec((_BLK, _D), lambda i: (i, 0)),
            pl.BlockSpec((_BLK, 1), lambda i: (i, 0)),
            pl.BlockSpec((1, _D), lambda i: (0, 0)),
        ],
        out_specs=pl.BlockSpec((_BLK, _D), lambda i: (i, 0)),
        out_shape=jax.ShapeDtypeStruct((_N, _D), jnp.float32),
    )(p, h, dinv2, b.reshape(1, _D))


def kernel(x, edge_index, edge_weights, W1, b1, W2, b2, W3, b3):
    src = edge_index[0]
    dst = edge_index[1]

    dinv2_pad, norm = _norm_kernel(src, dst, edge_weights)
    dinv2 = dinv2_pad[:_N].reshape(_N, 1)

    pad = _EP - _E
    zi = jnp.zeros((pad,), jnp.int32)
    src2d = jnp.concatenate([src, zi]).reshape(_EP // _C, _C)
    dst2d = jnp.concatenate([dst, zi]).reshape(_EP // _C, _C)
    norm2d = jnp.concatenate(
        [norm, jnp.zeros((pad,), jnp.float32)]).reshape(_EP // _C, _C)

    h1 = _matmul(x, W1)
    p1 = _agg_kernel(h1, src2d, dst2d, norm2d)
    h2 = _mid(p1, h1, dinv2, b1, W2)
    p2 = _agg_kernel(h2, src2d, dst2d, norm2d)
    h3 = _mid(p2, h2, dinv2, b2, W3)
    p3 = _agg_kernel(h3, src2d, dst2d, norm2d)
    return _final(p3, h3, dinv2, b3)


# speedup vs baseline: 1.8603x; 1.0092x over previous
import functools

import jax
import jax.numpy as jnp
from jax import lax
from jax.experimental import pallas as pl
from jax.experimental.pallas import tpu as pltpu
from jax.experimental.pallas import tpu_sc as plsc

_N = 10000
_E = 320000
_D = 128
_NC = 2
_NS = 16
_NW = _NC * _NS
_NPAD = 10240
_SEG = _NPAD // _NS
_EPS = _E // _NS
_EPT = _E // _NW
_C = 80
_NCH = 126
_EP = _NW * _NCH * _C
_NG = _NCH // 2
_NROW = _N // _NS

_mesh = plsc.VectorSubcoreMesh(core_axis_name="c", subcore_axis_name="s")
_sc_params = pltpu.CompilerParams(needs_layout_passes=False,
                                  use_tc_tiling_on_sc=False)


def _rsqrt16(x):
    i = plsc.bitcast(x, jnp.int32)
    i = jnp.int32(0x5F3759DF) - jnp.right_shift(i, 1)
    y = plsc.bitcast(i, jnp.float32)
    for _ in range(4):
        y = y * (jnp.float32(1.5) - jnp.float32(0.5) * x * y * y)
    return y


@functools.partial(
    pl.kernel,
    mesh=_mesh,
    out_type=(
        jax.ShapeDtypeStruct((_NPAD,), jnp.float32),
        jax.ShapeDtypeStruct((_E,), jnp.float32),
    ),
    scratch_types=[
        pltpu.VMEM((_EPS,), jnp.int32),
        pltpu.VMEM((_EPS,), jnp.float32),
        pltpu.VMEM((_NPAD,), jnp.float32),
        pltpu.VMEM((_SEG,), jnp.float32),
        pltpu.VMEM((_SEG,), jnp.float32),
        pltpu.VMEM((_NPAD,), jnp.float32),
        pltpu.VMEM((_EPT,), jnp.int32),
        pltpu.VMEM((_EPT,), jnp.float32),
        pltpu.VMEM_SHARED((_NS, _NPAD), jnp.float32),
        pltpu.VMEM_SHARED((_NPAD,), jnp.float32),
    ],
    compiler_params=_sc_params,
)
def _norm_kernel(src_hbm, dst_hbm, w_hbm, dinv2_hbm, norm_hbm,
                 dst_v, w_v, deg_v, acc_v, tmp_v, dinv_v, src_v, nrm_v,
                 slab_sh, dinv_sh):
    cid = lax.axis_index("c")
    sid = lax.axis_index("s")
    wid = cid * _NS + sid

    ebase = sid * _EPS
    pltpu.sync_copy(dst_hbm.at[pl.ds(ebase, _EPS)], dst_v)
    pltpu.sync_copy(w_hbm.at[pl.ds(ebase, _EPS)], w_v)

    def _zero(i, _):
        deg_v[pl.ds(i * 16, 16)] = jnp.zeros((16,), jnp.float32)
        return _
    lax.fori_loop(0, _NPAD // 16, _zero, None)

    def _deg(i, _):
        d16 = dst_v[pl.ds(i * 16, 16)]
        w16 = w_v[pl.ds(i * 16, 16)]
        plsc.addupdate_scatter(deg_v, [d16], w16)
        return _
    lax.fori_loop(0, _EPS // 16, _deg, None)

    pltpu.sync_copy(deg_v, slab_sh.at[sid])
    plsc.subcore_barrier()

    col0 = sid * _SEG
    pltpu.sync_copy(slab_sh.at[0, pl.ds(col0, _SEG)], acc_v)

    def _red(r, _):
        pltpu.sync_copy(slab_sh.at[r, pl.ds(col0, _SEG)], tmp_v)

        def _add(k, __):
            acc_v[pl.ds(k * 16, 16)] = (acc_v[pl.ds(k * 16, 16)]
                                        + tmp_v[pl.ds(k * 16, 16)])
            return __
        lax.fori_loop(0, _SEG // 16, _add, None)
        return _
    lax.fori_loop(1, _NS, _red, None)

    def _dinv(k, _):
        d = acc_v[pl.ds(k * 16, 16)] + jnp.float32(1.0)
        y = _rsqrt16(d)
        acc_v[pl.ds(k * 16, 16)] = y
        tmp_v[pl.ds(k * 16, 16)] = y * y
        return _
    lax.fori_loop(0, _SEG // 16, _dinv, None)

    pltpu.sync_copy(acc_v, dinv_sh.at[pl.ds(col0, _SEG)])

    @pl.when(cid == 0)
    def _():
        pltpu.sync_copy(tmp_v, dinv2_hbm.at[pl.ds(col0, _SEG)])

    plsc.subcore_barrier()
    pltpu.sync_copy(dinv_sh, dinv_v)

    nbase = wid * _EPT
    pltpu.sync_copy(src_hbm.at[pl.ds(nbase, _EPT)], src_v)
    pltpu.sync_copy(dst_hbm.at[pl.ds(nbase, _EPT)], dst_v.at[pl.ds(0, _EPT)])
    pltpu.sync_copy(w_hbm.at[pl.ds(nbase, _EPT)], w_v.at[pl.ds(0, _EPT)])

    def _nrm(i, _):
        s16 = src_v[pl.ds(i * 16, 16)]
        d16 = dst_v[pl.ds(i * 16, 16)]
        w16 = w_v[pl.ds(i * 16, 16)]
        a = plsc.load_gather(dinv_v, [s16])
        b = plsc.load_gather(dinv_v, [d16])
        nrm_v[pl.ds(i * 16, 16)] = a * w16 * b
        return _
    lax.fori_loop(0, _EPT // 16, _nrm, None)

    pltpu.sync_copy(nrm_v, norm_hbm.at[pl.ds(nbase, _EPT)])


@functools.partial(
    pl.kernel,
    mesh=_mesh,
    out_type=jax.ShapeDtypeStruct((_NC, _N, _D), jnp.float32),
    scratch_types=[
        pltpu.VMEM((_NCH, _C), jnp.int32),
        pltpu.VMEM((_NCH, _C), jnp.int32),
        pltpu.VMEM((_NCH, _C), jnp.float32),
        pltpu.VMEM((2, _C, _D), jnp.float32),
        pltpu.VMEM_SHARED((_N, _D), jnp.float32),
        pltpu.SemaphoreType.DMA,
        pltpu.SemaphoreType.DMA,
        pltpu.SemaphoreType.DMA,
        pltpu.SemaphoreType.DMA,
    ],
    compiler_params=_sc_params,
)
def _agg_kernel(h_hbm, src_hbm, dst_hbm, norm_hbm, out_hbm,
                src_v, dst_v, norm_v, rows_v, acc_sh, g0, g1, g2, g3):
    gsem = ((g0, g1), (g2, g3))
    _H = _C // 2
    cid = lax.axis_index("c")
    sid = lax.axis_index("s")
    wid = cid * _NS + sid
    row0 = wid * _NCH

    def _gissue(cc, b):
        pltpu.async_copy(h_hbm.at[src_v.at[cc, pl.ds(0, _H)]],
                         rows_v.at[b, pl.ds(0, _H)], gsem[b][0])
        pltpu.async_copy(h_hbm.at[src_v.at[cc, pl.ds(_H, _H)]],
                         rows_v.at[b, pl.ds(_H, _H)], gsem[b][1])

    def _gwait(cc, b):
        pltpu.make_async_copy(h_hbm.at[src_v.at[cc, pl.ds(0, _H)]],
                              rows_v.at[b, pl.ds(0, _H)], gsem[b][0]).wait()
        pltpu.make_async_copy(h_hbm.at[src_v.at[cc, pl.ds(_H, _H)]],
                              rows_v.at[b, pl.ds(_H, _H)], gsem[b][1]).wait()

    def _zr(i, _):
        for k in range(_D // 16):
            rows_v[0, i, pl.ds(k * 16, 16)] = jnp.zeros((16,), jnp.float32)
        return _
    lax.fori_loop(0, _C, _zr, None)

    r0 = sid * _NROW
    for j in range(_NROW // _C):
        pltpu.sync_copy(rows_v.at[0], acc_sh.at[pl.ds(r0 + j * _C, _C)])
    rem = _NROW % _C
    if rem:
        pltpu.sync_copy(rows_v.at[0, pl.ds(0, rem)],
                        acc_sh.at[pl.ds(r0 + (_NROW // _C) * _C, rem)])

    pltpu.sync_copy(src_hbm.at[pl.ds(row0, _NCH)], src_v)
    pltpu.sync_copy(dst_hbm.at[pl.ds(row0, _NCH)], dst_v)
    pltpu.sync_copy(norm_hbm.at[pl.ds(row0, _NCH)], norm_v)
    plsc.subcore_barrier()

    _gissue(0, 0)

    def _compute(c, b):
        def _scale(grp, __):
            n16 = norm_v[c, pl.ds(grp * 16, 16)]
            for l in range(16):
                e = grp * 16 + l
                nb = jnp.broadcast_to(n16[l], (16,))
                for k in range(_D // 16):
                    rows_v[b, e, pl.ds(k * 16, 16)] = (
                        rows_v[b, e, pl.ds(k * 16, 16)] * nb)
            return __
        lax.fori_loop(0, _C // 16, _scale, None)

    def _group(g, _):
        for u in range(2):
            c = g * 2 + u
            b = u
            if u == 0:
                _gissue(c + 1, 1)
            else:
                @pl.when(g < _NG - 1)
                def _():
                    _gissue(c + 1, 0)
            _gwait(c, b)
            _compute(c, b)
            pltpu.sync_copy(rows_v.at[b], acc_sh.at[dst_v.at[c]], add=True)
        return _
    lax.fori_loop(0, _NG, _group, None)

    plsc.subcore_barrier()
    pltpu.sync_copy(acc_sh.at[pl.ds(r0, _NROW)],
                    out_hbm.at[cid, pl.ds(r0, _NROW)])


_BLK = 400


def _mm_body(x_ref, w_ref, o_ref):
    o_ref[...] = jnp.dot(x_ref[...], w_ref[...],
                         preferred_element_type=jnp.float32)


def _matmul(x, w):
    return pl.pallas_call(
        _mm_body,
        grid=(_N // _BLK,),
        in_specs=[
            pl.BlockSpec((_BLK, _D), lambda i: (i, 0)),
            pl.BlockSpec((_D, _D), lambda i: (0, 0)),
        ],
        out_specs=pl.BlockSpec((_BLK, _D), lambda i: (i, 0)),
        out_shape=jax.ShapeDtypeStruct((_N, _D), jnp.float32),
    )(x, w)


def _mid_body(p_ref, h_ref, d_ref, b_ref, w_ref, o_ref):
    agg = p_ref[0] + p_ref[1] + d_ref[...] * h_ref[...] + b_ref[...]
    a = jnp.maximum(agg, 0.0)
    o_ref[...] = jnp.dot(a, w_ref[...], preferred_element_type=jnp.float32)


def _mid(p, h, dinv2, b, w):
    return pl.pallas_call(
        _mid_body,
        grid=(_N // _BLK,),
        in_specs=[
            pl.BlockSpec((_NC, _BLK, _D), lambda i: (0, i, 0)),
            pl.BlockSpec((_BLK, _D), lambda i: (i, 0)),
            pl.BlockSpec((_BLK, 1), lambda i: (i, 0)),
            pl.BlockSpec((1, _D), lambda i: (0, 0)),
            pl.BlockSpec((_D, _D), lambda i: (0, 0)),
        ],
        out_specs=pl.BlockSpec((_BLK, _D), lambda i: (i, 0)),
        out_shape=jax.ShapeDtypeStruct((_N, _D), jnp.float32),
    )(p, h, dinv2, b.reshape(1, _D), w)


def _final_body(p_ref, h_ref, d_ref, b_ref, o_ref):
    agg = p_ref[0] + p_ref[1] + d_ref[...] * h_ref[...] + b_ref[...]
    o_ref[...] = jax.nn.sigmoid(agg)


def _final(p, h, dinv2, b):
    return pl.pallas_call(
        _final_body,
        grid=(_N // _BLK,),
        in_specs=[
            pl.BlockSpec((_NC, _BLK, _D), lambda i: (0, i, 0)),
            pl.BlockSpec((_BLK, _D), lambda i: (i, 0)),
            pl.BlockSpec((_BLK, 1), lambda i: (i, 0)),
            pl.BlockSpec((1, _D), lambda i: (0, 0)),
        ],
        out_specs=pl.BlockSpec((_BLK, _D), lambda i: (i, 0)),
        out_shape=jax.ShapeDtypeStruct((_N, _D), jnp.float32),
    )(p, h, dinv2, b.reshape(1, _D))


def kernel(x, edge_index, edge_weights, W1, b1, W2, b2, W3, b3):
    src = edge_index[0]
    dst = edge_index[1]

    dinv2_pad, norm = _norm_kernel(src, dst, edge_weights)
    dinv2 = dinv2_pad[:_N].reshape(_N, 1)

    pad = _EP - _E
    zi = jnp.zeros((pad,), jnp.int32)
    src2d = jnp.concatenate([src, zi]).reshape(_EP // _C, _C)
    dst2d = jnp.concatenate([dst, zi]).reshape(_EP // _C, _C)
    norm2d = jnp.concatenate(
        [norm, jnp.zeros((pad,), jnp.float32)]).reshape(_EP // _C, _C)

    h1 = _matmul(x, W1)
    p1 = _agg_kernel(h1, src2d, dst2d, norm2d)
    h2 = _mid(p1, h1, dinv2, b1, W2)
    p2 = _agg_kernel(h2, src2d, dst2d, norm2d)
    h3 = _mid(p2, h2, dinv2, b2, W3)
    p3 = _agg_kernel(h3, src2d, dst2d, norm2d)
    return _final(p3, h3, dinv2, b3)
